# baseline (speedup 1.0000x reference)
import functools
import math

import jax
import jax.numpy as jnp
from jax import lax
from jax.experimental import pallas as pl
from jax.experimental.pallas import tpu as pltpu
from jax.experimental.pallas import tpu_sc as plsc

D_MODEL = 64
SCALE = math.sqrt(D_MODEL)
NUM_CORES = 2
NUM_SUBCORES = 16
NUM_WORKERS = NUM_CORES * NUM_SUBCORES
LANES = 16

VPAD = 1000064
SEQ = 200
BATCH = 4096
BSLICE = BATCH // NUM_WORKERS
NSTEP = SEQ // 2


def _repack_body(lutT_ref, lp_ref):
    lp_ref[:, 0:D_MODEL] = lutT_ref[...].T * SCALE


def _repack(lutT):
    return pl.pallas_call(
        _repack_body,
        grid=(VPAD // 128,),
        in_specs=[pl.BlockSpec((D_MODEL, 128), lambda i: (0, i))],
        out_specs=pl.BlockSpec((128, 128), lambda i: (i, 0)),
        out_shape=jax.ShapeDtypeStruct((VPAD, 128), jnp.float32),
    )(lutT)


def _gather_body(xT_hbm, lp_hbm, out_hbm, idx_v, rows0, rows1, t0, t1,
                 gsem0, gsem1, wsem0, wsem1):
    wid = lax.axis_index("s") * NUM_CORES + lax.axis_index("c")
    b0 = wid * BSLICE

    rows = (rows0, rows1)
    tbuf = (t0, t1)
    gsem = (gsem0, gsem1)
    wsem = (wsem0, wsem1)

    pltpu.sync_copy(xT_hbm.at[:, pl.ds(b0, BSLICE)], idx_v)

    def gather(i, b):
        for r in range(2):
            pltpu.async_copy(
                lp_hbm.at[idx_v.at[2 * i + r]],
                rows[b].at[pl.ds(r * BSLICE, BSLICE)],
                gsem[b],
            )

    def gather_wait(b):
        for r in range(2):
            pltpu.make_async_copy(
                lp_hbm.at[idx_v.at[r]],
                rows[b].at[pl.ds(r * BSLICE, BSLICE)],
                gsem[b],
            ).wait()

    def permute(b):
        rv = rows[b]
        tb = tbuf[b]
        for s_ in range(2):
            rbase = s_ * BSLICE

            @plsc.parallel_loop(0, D_MODEL, unroll=2)
            def _(d):
                dvec = jnp.zeros((LANES,), jnp.int32) + d
                for bg in range(BSLICE // LANES):
                    ridx = rbase + bg * LANES + lax.iota(jnp.int32, LANES)
                    vec = plsc.load_gather(rv, [ridx, dvec])
                    tb[s_, d, pl.ds(bg * LANES, LANES)] = vec

    def write(i, b):
        pltpu.async_copy(
            tbuf[b], out_hbm.at[pl.ds(2 * i, 2), :, pl.ds(b0, BSLICE)], wsem[b]
        )

    def write_wait(b):
        pltpu.make_async_copy(
            tbuf[b], out_hbm.at[pl.ds(0, 2), :, pl.ds(b0, BSLICE)], wsem[b]
        ).wait()

    def step(i, b, first, last):
        nb = 1 - b
        gather_wait(b)
        if not last:
            gather(i + 1, nb)
        if not first:
            write_wait(b)
        permute(b)
        write(i, b)

    gather(0, 0)
    step(0, 0, first=True, last=False)
    step(1, 1, first=True, last=False)

    def pair(p, c):
        i = 2 * p
        step(i, 0, first=False, last=False)
        step(i + 1, 1, first=False, last=False)
        return c

    lax.fori_loop(1, NSTEP // 2 - 1, pair, 0)

    step(NSTEP - 2, 0, first=False, last=False)
    step(NSTEP - 1, 1, first=False, last=True)

    write_wait(0)
    write_wait(1)


def _gather(xT, lp):
    mesh = plsc.VectorSubcoreMesh(
        core_axis_name="c",
        subcore_axis_name="s",
        num_cores=NUM_CORES,
        num_subcores=NUM_SUBCORES,
    )
    return pl.kernel(
        _gather_body,
        out_type=jax.ShapeDtypeStruct((SEQ, D_MODEL, BATCH), jnp.float32),
        mesh=mesh,
        scratch_types=[
            pltpu.VMEM((SEQ, BSLICE), jnp.int32),
            pltpu.VMEM((2 * BSLICE, 128), jnp.float32),
            pltpu.VMEM((2 * BSLICE, 128), jnp.float32),
            pltpu.VMEM((2, D_MODEL, BSLICE), jnp.float32),
            pltpu.VMEM((2, D_MODEL, BSLICE), jnp.float32),
            pltpu.SemaphoreType.DMA,
            pltpu.SemaphoreType.DMA,
            pltpu.SemaphoreType.DMA,
            pltpu.SemaphoreType.DMA,
        ],
        compiler_params=pltpu.CompilerParams(
            use_tc_tiling_on_sc=True, needs_layout_passes=False
        ),
    )(xT, lp)


def kernel(x, lut):
    lp = _repack(lut.T)
    outT = _gather(x.T, lp)
    return jnp.transpose(outT, (2, 0, 1))


# speedup vs baseline: 5.2627x; 5.2627x over previous
import functools
import math

import jax
import jax.numpy as jnp
from jax import lax
from jax.experimental import pallas as pl
from jax.experimental.pallas import tpu as pltpu
from jax.experimental.pallas import tpu_sc as plsc

D_MODEL = 64
SCALE = math.sqrt(D_MODEL)
NUM_CORES = 2
NUM_SUBCORES = 16
NUM_WORKERS = NUM_CORES * NUM_SUBCORES
LANES = 16

VOCAB = 1000000
RBLK = 16000
SEQ = 200
BATCH = 4096
BSLICE = BATCH // NUM_WORKERS
NSTEP = SEQ // 2


def _repack_body(lutT_ref, lp_ref):
    lp_ref[:, 0:D_MODEL] = lutT_ref[...].T * SCALE


def _repack(lutT):
    return pl.pallas_call(
        _repack_body,
        grid=((VOCAB + RBLK - 1) // RBLK,),
        in_specs=[pl.BlockSpec((D_MODEL, RBLK), lambda i: (0, i))],
        out_specs=pl.BlockSpec((RBLK, 128), lambda i: (i, 0)),
        out_shape=jax.ShapeDtypeStruct((VOCAB, 128), jnp.float32),
    )(lutT)


def _gather_body(xT_hbm, lp_hbm, out_hbm, idx_v, rows0, rows1, t0, t1,
                 gsem0, gsem1, wsem0, wsem1):
    wid = lax.axis_index("s") * NUM_CORES + lax.axis_index("c")
    b0 = wid * BSLICE

    rows = (rows0, rows1)
    tbuf = (t0, t1)
    gsem = (gsem0, gsem1)
    wsem = (wsem0, wsem1)

    pltpu.sync_copy(xT_hbm.at[:, pl.ds(b0, BSLICE)], idx_v)

    def gather(i, b):
        for r in range(2):
            pltpu.async_copy(
                lp_hbm.at[idx_v.at[2 * i + r]],
                rows[b].at[pl.ds(r * BSLICE, BSLICE)],
                gsem[b],
            )

    def gather_wait(b):
        for r in range(2):
            pltpu.make_async_copy(
                lp_hbm.at[idx_v.at[r]],
                rows[b].at[pl.ds(r * BSLICE, BSLICE)],
                gsem[b],
            ).wait()

    def permute(b):
        rv = rows[b]
        tb = tbuf[b]
        for s_ in range(2):
            rbase = s_ * BSLICE

            @plsc.parallel_loop(0, D_MODEL, unroll=2)
            def _(d):
                dvec = jnp.zeros((LANES,), jnp.int32) + d
                for bg in range(BSLICE // LANES):
                    ridx = rbase + bg * LANES + lax.iota(jnp.int32, LANES)
                    vec = plsc.load_gather(rv, [ridx, dvec])
                    tb[s_, d, pl.ds(bg * LANES, LANES)] = vec

    def write(i, b):
        pltpu.async_copy(
            tbuf[b], out_hbm.at[pl.ds(2 * i, 2), :, pl.ds(b0, BSLICE)], wsem[b]
        )

    def write_wait(b):
        pltpu.make_async_copy(
            tbuf[b], out_hbm.at[pl.ds(0, 2), :, pl.ds(b0, BSLICE)], wsem[b]
        ).wait()

    def step(i, b, first, last):
        nb = 1 - b
        gather_wait(b)
        if not last:
            gather(i + 1, nb)
        if not first:
            write_wait(b)
        permute(b)
        write(i, b)

    gather(0, 0)
    step(0, 0, first=True, last=False)
    step(1, 1, first=True, last=False)

    def pair(p, c):
        i = 2 * p
        step(i, 0, first=False, last=False)
        step(i + 1, 1, first=False, last=False)
        return c

    lax.fori_loop(1, NSTEP // 2 - 1, pair, 0)

    step(NSTEP - 2, 0, first=False, last=False)
    step(NSTEP - 1, 1, first=False, last=True)

    write_wait(0)
    write_wait(1)


def _gather(xT, lp):
    mesh = plsc.VectorSubcoreMesh(
        core_axis_name="c",
        subcore_axis_name="s",
        num_cores=NUM_CORES,
        num_subcores=NUM_SUBCORES,
    )
    return pl.kernel(
        _gather_body,
        out_type=jax.ShapeDtypeStruct((SEQ, D_MODEL, BATCH), jnp.float32),
        mesh=mesh,
        scratch_types=[
            pltpu.VMEM((SEQ, BSLICE), jnp.int32),
            pltpu.VMEM((2 * BSLICE, 128), jnp.float32),
            pltpu.VMEM((2 * BSLICE, 128), jnp.float32),
            pltpu.VMEM((2, D_MODEL, BSLICE), jnp.float32),
            pltpu.VMEM((2, D_MODEL, BSLICE), jnp.float32),
            pltpu.SemaphoreType.DMA,
            pltpu.SemaphoreType.DMA,
            pltpu.SemaphoreType.DMA,
            pltpu.SemaphoreType.DMA,
        ],
        compiler_params=pltpu.CompilerParams(
            use_tc_tiling_on_sc=True, needs_layout_passes=False
        ),
    )(xT, lp)


def kernel(x, lut):
    lp = _repack(lut.T)
    outT = _gather(x.T, lp)
    return jnp.transpose(outT, (2, 0, 1))
